# Initial kernel scaffold; baseline (speedup 1.0000x reference)
import jax
import jax.numpy as jnp
from jax.experimental import pallas as pl


def kernel(x, edge_index, W1, b1, W2, b2):
    raise NotImplementedError("write your pallas kernel here")


# speedup vs baseline: 13.7604x; 13.7604x over previous
import functools

import jax
import jax.numpy as jnp
from jax import lax
from jax.experimental import pallas as pl
from jax.experimental.pallas import tpu as pltpu
from jax.experimental.pallas import tpu_sc as plsc

N = 10000
E = 320000
D = 128

NC = 2
NS = 16
NW = NC * NS
NPAD = 10240
RPT = NPAD // NS
EPT = E // NW
C = 80
NCHUNK = EPT // C

_mesh = plsc.VectorSubcoreMesh(core_axis_name="c", subcore_axis_name="s",
                               num_cores=NC, num_subcores=NS)

def _zero16():
    return jnp.zeros((16,), jnp.float32)


def _one16():
    return jnp.ones((16,), jnp.float32)


def _tile_ids():
    c = lax.axis_index("c")
    s = lax.axis_index("s")
    return c, s, c * NS + s


@functools.partial(
    pl.kernel,
    out_type=jax.ShapeDtypeStruct((NC, NPAD), jnp.float32),
    mesh=_mesh,
    scratch_types=[
        pltpu.VMEM_SHARED((NPAD,), jnp.float32),
        pltpu.VMEM((C,), jnp.int32),
        pltpu.VMEM((C,), jnp.float32),
        pltpu.VMEM((RPT,), jnp.float32),
    ],
)
def _sc_deg(dst_hbm, deg_out, deg_sh, idx_v, ones_v, zv):
    c, s, wid = _tile_ids()

    def fill_z(i, _):
        zv[pl.ds(i * 16, 16)] = _zero16()
        return 0
    lax.fori_loop(0, RPT // 16, fill_z, 0)
    for i in range(C // 16):
        ones_v[pl.ds(i * 16, 16)] = _one16()

    pltpu.sync_copy(zv, deg_sh.at[pl.ds(s * RPT, RPT)])
    plsc.subcore_barrier()

    def step(j, _):
        base = wid * EPT + j * C
        pltpu.sync_copy(dst_hbm.at[pl.ds(base, C)], idx_v)
        pltpu.sync_copy(ones_v, deg_sh.at[idx_v], add=True)
        return 0
    lax.fori_loop(0, NCHUNK, step, 0)
    plsc.subcore_barrier()

    pltpu.sync_copy(deg_sh.at[pl.ds(s * RPT, RPT)],
                    deg_out.at[c, pl.ds(s * RPT, RPT)])


@functools.partial(
    pl.kernel,
    out_type=jax.ShapeDtypeStruct((NC, NPAD, D), jnp.float32),
    mesh=_mesh,
    scratch_types=[
        pltpu.VMEM_SHARED((NPAD, D), jnp.float32),
        pltpu.VMEM((C,), jnp.int32),
        pltpu.VMEM((C,), jnp.int32),
        pltpu.VMEM((C, D), jnp.float32),
        pltpu.VMEM((128, D), jnp.float32),
        pltpu.SemaphoreType.DMA,
    ],
)
def _sc_msg(hp_hbm, src_hbm, dst_hbm, acc_out, acc_sh, sidx, didx, rows, zrow, sem):
    c, s, wid = _tile_ids()

    def fill_z(i, _):
        for j in range(D // 16):
            zrow[i, pl.ds(j * 16, 16)] = _zero16()
        return 0
    lax.fori_loop(0, 128, fill_z, 0)

    def init(k, _):
        pltpu.sync_copy(zrow, acc_sh.at[pl.ds(s * RPT + k * 128, 128)])
        return 0
    lax.fori_loop(0, RPT // 128, init, 0)
    plsc.subcore_barrier()

    def step(j, _):
        base = wid * EPT + j * C
        pltpu.sync_copy(src_hbm.at[pl.ds(base, C)], sidx)
        pltpu.sync_copy(dst_hbm.at[pl.ds(base, C)], didx)
        pltpu.async_copy(hp_hbm.at[sidx], rows, sem).wait()
        pltpu.sync_copy(rows, acc_sh.at[didx], add=True)
        return 0
    lax.fori_loop(0, NCHUNK, step, 0)
    plsc.subcore_barrier()

    pltpu.sync_copy(acc_sh.at[pl.ds(s * RPT, RPT)],
                    acc_out.at[c, pl.ds(s * RPT, RPT)])


_BLK = 1024
_GRID = (N + _BLK - 1) // _BLK


def _dis_col(deg_ref):
    return lax.rsqrt(deg_ref[0, :] + deg_ref[1, :] + 1.0)[:, None]


def _tc_pre_body(x_ref, w_ref, deg_ref, o_ref):
    h = jnp.dot(x_ref[:], w_ref[:], preferred_element_type=jnp.float32)
    o_ref[:] = h * _dis_col(deg_ref)


def _tc_mid_body(acc_ref, hp_ref, deg_ref, b_ref, w_ref, o_ref):
    dis = _dis_col(deg_ref)
    t = dis * (acc_ref[0] + acc_ref[1] + hp_ref[:]) + b_ref[:]
    t = jnp.maximum(t, 0.0)
    o_ref[:] = jnp.dot(t, w_ref[:], preferred_element_type=jnp.float32) * dis


def _tc_post_body(acc_ref, hp_ref, deg_ref, b_ref, o_ref):
    dis = _dis_col(deg_ref)
    o_ref[:] = jnp.maximum(dis * (acc_ref[0] + acc_ref[1] + hp_ref[:]) + b_ref[:],
                           0.0)


_x_spec = pl.BlockSpec((_BLK, D), lambda j: (j, 0))
_w_spec = pl.BlockSpec((D, D), lambda j: (0, 0))
_deg_spec = pl.BlockSpec((NC, _BLK), lambda j: (0, j))
_acc_spec = pl.BlockSpec((NC, _BLK, D), lambda j: (0, j, 0))
_b_spec = pl.BlockSpec((1, D), lambda j: (0, 0))
_out_sds = jax.ShapeDtypeStruct((N, D), jnp.float32)

_tc_pre = pl.pallas_call(
    _tc_pre_body, grid=(_GRID,),
    in_specs=[_x_spec, _w_spec, _deg_spec],
    out_specs=_x_spec, out_shape=_out_sds)

_tc_mid = pl.pallas_call(
    _tc_mid_body, grid=(_GRID,),
    in_specs=[_acc_spec, _x_spec, _deg_spec, _b_spec, _w_spec],
    out_specs=_x_spec, out_shape=_out_sds)

_tc_post = pl.pallas_call(
    _tc_post_body, grid=(_GRID,),
    in_specs=[_acc_spec, _x_spec, _deg_spec, _b_spec],
    out_specs=_x_spec, out_shape=_out_sds)


@jax.jit
def kernel(x, edge_index, W1, b1, W2, b2):
    src = edge_index[0].astype(jnp.int32)
    dst = edge_index[1].astype(jnp.int32)
    b1r = b1.reshape(1, D)
    b2r = b2.reshape(1, D)

    deg2 = _sc_deg(dst)
    hp1 = _tc_pre(x, W1, deg2)
    acc1 = _sc_msg(hp1, src, dst)
    hp2 = _tc_mid(acc1, hp1, deg2, b1r, W2)
    acc2 = _sc_msg(hp2, src, dst)
    return _tc_post(acc2, hp2, deg2, b2r)
